# baseline (speedup 1.0000x reference)
import jax
import jax.numpy as jnp
from jax import lax
from jax.experimental import pallas as pl
from jax.experimental.pallas import tpu as pltpu
from jax.experimental.pallas import tpu_sc as plsc

B = 1024
V = 6890
F = 13776
J = 32
NC = 2
NS = 16
NW = NC * NS
BPW = B // NW
NENT = BPW * J * 3
CHUNK = 128
NCHUNK = 2 * NENT // CHUNK
M8 = B * V * 3 // 8
FROWS = F * 3 // 16

CP = pltpu.CompilerParams(use_tc_tiling_on_sc=False, needs_layout_passes=False)


def _body(verts_hbm, faces_hbm, fids_hbm, bcs_hbm, out_hbm,
          fid_v, fq_v, fbuf_v, bcs_v, wbuf_v, idx_v, off_v, g_v, o_v, sem):
    wid = lax.axis_index("s") * NC + lax.axis_index("c")
    base_b = wid * BPW
    lane = lax.iota(jnp.int32, 16)

    pltpu.sync_copy(fids_hbm, fid_v)
    pltpu.sync_copy(bcs_hbm, bcs_v)

    for p in range(2):
        fidv = fid_v[pl.ds(p * 16, 16)]
        q = (fidv * 3) // 16
        fq_v[pl.ds(p * 16, 16)] = q
        fq_v[pl.ds(32 + p * 16, 16)] = jnp.minimum(q + 1, FROWS - 1)
    pltpu.async_copy(faces_hbm.at[fq_v], fbuf_v, sem).wait()

    def build(s, carry):
        k = s // 64
        p = s % 2
        b_local = (s % 64) // 2
        jj = 16 * p + lane
        fidv = plsc.load_gather(fid_v, [jj])
        w = (fidv * 3) % 16 + k
        frow = jnp.where(w < 16, jj, 32 + jj)
        fcol = w % 16
        vidv = plsc.load_gather(fbuf_v, [frow, fcol])
        o = ((base_b + b_local) * V + vidv) * 3
        r = o // 8
        idx_v[s // 8, pl.ds(pl.multiple_of((s % 8) * 16, 16), 16)] = r
        idx_v[24 + s // 8, pl.ds(pl.multiple_of((s % 8) * 16, 16), 16)] = (
            jnp.minimum(r + 1, M8 - 1))
        off_v[pl.ds(pl.multiple_of(s * 16, 16), 16)] = o % 8
        return carry

    lax.fori_loop(0, NENT // 16, build, 0)

    for k in range(3):
        for v6 in range(6):
            f = v6 * 16 + lane
            wvec = plsc.load_gather(bcs_v, [(f // 3) * 3 + k])
            wbuf_v[pl.ds(k * 96 + v6 * 16, 16)] = wvec

    def fire(t, carry):
        pltpu.make_async_copy(
            verts_hbm.at[idx_v.at[t]],
            g_v.at[pl.ds(pl.multiple_of(t * CHUNK, CHUNK), CHUNK)], sem).start()
        return carry

    lax.fori_loop(0, NCHUNK, fire, 0)

    def drain(t, carry):
        pltpu.make_async_copy(
            verts_hbm.at[idx_v.at[t]],
            g_v.at[pl.ds(pl.multiple_of(t * CHUNK, CHUNK), CHUNK)], sem).wait()
        return carry

    lax.fori_loop(0, NCHUNK, drain, 0)

    def comp(v, carry):
        o = v * 16 + lane
        b = o // 96
        j3c = o % 96
        jj = j3c // 3
        cc = j3c % 3
        woff = (v * 16) % 96
        acc = jnp.zeros((16,), jnp.float32)
        for k in range(3):
            i = k * 1024 + b * 32 + jj
            offs = plsc.load_gather(off_v, [i])
            w = offs + cc
            grow = jnp.where(w < 8, i, NENT + i)
            gcol = w % 8
            gk = plsc.load_gather(g_v, [grow, gcol])
            wk = wbuf_v[pl.ds(pl.multiple_of(k * 96 + woff, 8), 16)]
            acc = acc + wk * gk
        o_v[pl.ds(pl.multiple_of(v * 16, 16), 16)] = acc
        return carry

    lax.fori_loop(0, NENT // 16, comp, 0)

    pltpu.sync_copy(o_v, out_hbm.at[pl.ds(pl.multiple_of(wid * NENT, 16), NENT)])


@jax.jit
def _joints_sc(verts8, faces16, face_ids, bcs_flat):
    mesh = plsc.VectorSubcoreMesh(core_axis_name="c", subcore_axis_name="s")
    fn = pl.kernel(
        _body,
        out_type=jax.ShapeDtypeStruct((B * J * 3,), jnp.float32),
        mesh=mesh,
        scratch_types=[
            pltpu.VMEM((J,), jnp.int32),
            pltpu.VMEM((2 * J,), jnp.int32),
            pltpu.VMEM((2 * J, 16), jnp.int32),
            pltpu.VMEM((J * 3,), jnp.float32),
            pltpu.VMEM((3 * J * 3,), jnp.float32),
            pltpu.VMEM((NCHUNK, CHUNK), jnp.int32),
            pltpu.VMEM((NENT,), jnp.int32),
            pltpu.VMEM((2 * NENT, 8), jnp.float32),
            pltpu.VMEM((NENT,), jnp.float32),
            pltpu.SemaphoreType.DMA,
        ],
        compiler_params=CP,
    )
    return fn(verts8, faces16, face_ids, bcs_flat)


def kernel(vertices, faces, face_ids, bcs):
    verts8 = vertices.reshape(M8, 8)
    faces16 = faces.reshape(FROWS, 16)
    out = _joints_sc(verts8, faces16, face_ids.astype(jnp.int32),
                     bcs.reshape(J * 3))
    return out.reshape(B, J, 3)


# speedup vs baseline: 744.9179x; 744.9179x over previous
import jax
import jax.numpy as jnp
from jax import lax
from jax.experimental import pallas as pl
from jax.experimental.pallas import tpu as pltpu
from jax.experimental.pallas import tpu_sc as plsc

B = 1024
V = 6890
F = 13776
J = 32
NC = 2
NS = 16
NW = NC * NS
PPW = 3
FWPAD = F * 3 + 16

CP = pltpu.CompilerParams(use_tc_tiling_on_sc=True, needs_layout_passes=False)


def _body(vT_hbm, faces_hbm, fids_hbm, bcs_hbm, out_hbm,
          fid_v, bcs_v, fwin_v, vbuf_v, obuf_v, semf, semv):
    wid = lax.axis_index("s") * NC + lax.axis_index("c")
    lane = lax.iota(jnp.int32, 16)

    pltpu.sync_copy(fids_hbm, fid_v)
    pltpu.sync_copy(bcs_hbm, bcs_v)
    f0 = fid_v[pl.ds(0, 16)]
    f1 = fid_v[pl.ds(16, 16)]
    b_vecs = [bcs_v[pl.ds(16 * i, 16)] for i in range(6)]

    ps = [3 * wid + e for e in range(PPW)]

    fids = []
    fcopies = []
    for e in range(PPW):
        j = ps[e] % 32
        fsel = jnp.where(j < 16, f0, f1)
        fid_j = jnp.sum(jnp.where(lane == j % 16, fsel, 0))
        fids.append(fid_j)
        a = pl.multiple_of((fid_j * 3 // 16) * 16, 16)
        fcopies.append(
            pltpu.async_copy(faces_hbm.at[pl.ds(a, 32)],
                             fwin_v.at[pl.ds(e * 32, 32)], semf))

    vcopies = []
    wgts = []
    for e in range(PPW):
        fcopies[e].wait()
        p = ps[e]
        c = p // 32
        j = p % 32
        woff = (fids[e] * 3) % 16
        lo = fwin_v[pl.ds(e * 32, 16)]
        hi = fwin_v[pl.ds(e * 32 + 16, 16)]
        for k in range(3):
            t = woff + k
            vsel = jnp.where(t < 16, lo, hi)
            vid = jnp.sum(jnp.where(lane == t % 16, vsel, 0))
            vcopies.append(
                pltpu.async_copy(vT_hbm.at[c, vid],
                                 vbuf_v.at[pl.ds((e * 3 + k) * B, B)], semv))
            tt = j * 3 + k
            vi = tt // 16
            bsel = b_vecs[5]
            for n in range(4, -1, -1):
                bsel = jnp.where(vi == n, b_vecs[n], bsel)
            wgts.append(jnp.sum(jnp.where(lane == tt % 16, bsel, 0.0)))

    for cpy in vcopies:
        cpy.wait()

    for e in range(PPW):
        w0, w1, w2 = wgts[3 * e], wgts[3 * e + 1], wgts[3 * e + 2]

        def comp(m, carry, e=e, w0=w0, w1=w1, w2=w2):
            off = pl.multiple_of(m * 16, 16)
            acc = (w0 * vbuf_v[pl.ds((3 * e) * B + off, 16)]
                   + w1 * vbuf_v[pl.ds((3 * e + 1) * B + off, 16)]
                   + w2 * vbuf_v[pl.ds((3 * e + 2) * B + off, 16)])
            obuf_v[pl.ds(e * B + off, 16)] = acc
            return carry

        lax.fori_loop(0, B // 16, comp, 0)

    for e in range(PPW):
        p = ps[e]
        pltpu.sync_copy(obuf_v.at[pl.ds(e * B, B)],
                        out_hbm.at[p // 32, p % 32])


@jax.jit
def _joints_sc(vT, faces_flat, face_ids, bcs_flat):
    mesh = plsc.VectorSubcoreMesh(core_axis_name="c", subcore_axis_name="s")
    fn = pl.kernel(
        _body,
        out_type=jax.ShapeDtypeStruct((3, J, B), jnp.float32),
        mesh=mesh,
        scratch_types=[
            pltpu.VMEM((J,), jnp.int32),
            pltpu.VMEM((J * 3,), jnp.float32),
            pltpu.VMEM((PPW * 32,), jnp.int32),
            pltpu.VMEM((9 * B,), jnp.float32),
            pltpu.VMEM((PPW * B,), jnp.float32),
            pltpu.SemaphoreType.DMA,
            pltpu.SemaphoreType.DMA,
        ],
        compiler_params=CP,
    )
    return fn(vT, faces_flat, face_ids, bcs_flat)


def kernel(vertices, faces, face_ids, bcs):
    vT = jnp.transpose(vertices, (2, 1, 0))
    faces_pad = jnp.pad(faces.reshape(F * 3), (0, FWPAD - F * 3))
    out_t = _joints_sc(vT, faces_pad, face_ids.astype(jnp.int32),
                       bcs.reshape(J * 3))
    return jnp.transpose(out_t, (2, 1, 0))
